# baseline (device time: 39320 ns/iter reference)
import functools

import jax
import jax.numpy as jnp
from jax import lax
from jax.experimental import pallas as pl
from jax.experimental.pallas import tpu as pltpu

N_DEV = 32
N_ROUNDS = 5


def kernel(x, router_W, route_idx, expert_W):
    n_tok, d_model = x.shape
    n_experts = router_W.shape[1]
    e_per, _, d_out = expert_W.shape

    def body(x_ref, rw_ref, idx_ref, ew_ref, out_ref,
             send_buf, recv_buf, send_sem, recv_sems):
        my = lax.axis_index("i")

        barrier_sem = pltpu.get_barrier_semaphore()
        for r in range(N_ROUNDS):
            partner = my ^ (1 << r)
            pl.semaphore_signal(
                barrier_sem, inc=1,
                device_id=(partner,), device_id_type=pl.DeviceIdType.MESH,
            )
        pl.semaphore_wait(barrier_sem, N_ROUNDS)

        xv = x_ref[...]
        scores = jnp.dot(xv, rw_ref[...],
                         preferred_element_type=jnp.float32)
        m = jnp.max(scores, axis=-1, keepdims=True)
        p = jnp.exp(scores - m)
        p = p / jnp.sum(p, axis=-1, keepdims=True)

        idx0 = idx_ref[:, 0:1]
        idx1 = idx_ref[:, 1:2]
        cols = lax.broadcasted_iota(jnp.int32, p.shape, 1)
        g0 = jnp.sum(jnp.where(cols == idx0, p, 0.0), axis=-1, keepdims=True)
        g1 = jnp.sum(jnp.where(cols == idx1, p, 0.0), axis=-1, keepdims=True)
        gs = g0 + g1

        acc = jnp.zeros((n_tok, d_out), jnp.float32)
        for k in range(e_per):
            e_glob = my * e_per + k
            w = (jnp.where(idx0 == e_glob, g0, 0.0)
                 + jnp.where(idx1 == e_glob, g1, 0.0)) / gs
            acc = acc + jnp.dot(xv * w, ew_ref[k],
                                preferred_element_type=jnp.float32)

        for r in range(N_ROUNDS):
            partner = my ^ (1 << r)
            send_buf[...] = acc
            rdma = pltpu.make_async_remote_copy(
                src_ref=send_buf,
                dst_ref=recv_buf.at[r],
                send_sem=send_sem,
                recv_sem=recv_sems.at[r],
                device_id=(partner,),
                device_id_type=pl.DeviceIdType.MESH,
            )
            rdma.start()
            rdma.wait()
            acc = acc + recv_buf[r]

        out_ref[...] = acc

        @functools.partial(pl.run_scoped,
                           exit_sem=pltpu.SemaphoreType.REGULAR)
        def _(exit_sem):
            for r in range(N_ROUNDS):
                partner = my ^ (1 << r)
                pl.semaphore_signal(
                    exit_sem, inc=1,
                    device_id=(partner,), device_id_type=pl.DeviceIdType.MESH,
                )
            pl.semaphore_wait(exit_sem, N_ROUNDS)

    return pl.pallas_call(
        body,
        out_shape=jax.ShapeDtypeStruct((n_tok, d_out), jnp.float32),
        in_specs=[pl.BlockSpec(memory_space=pltpu.VMEM)] * 4,
        out_specs=pl.BlockSpec(memory_space=pltpu.VMEM),
        scratch_shapes=[
            pltpu.VMEM((n_tok, d_out), jnp.float32),
            pltpu.VMEM((N_ROUNDS, n_tok, d_out), jnp.float32),
            pltpu.SemaphoreType.DMA,
            pltpu.SemaphoreType.DMA((N_ROUNDS,)),
        ],
        compiler_params=pltpu.CompilerParams(collective_id=0),
    )(x, router_W, route_idx, expert_W)
